# baseline (speedup 1.0000x reference)
import functools

import jax
import jax.numpy as jnp
from jax import lax
from jax.experimental import pallas as pl
from jax.experimental.pallas import tpu as pltpu
from jax.experimental.pallas import tpu_sc as plsc

BATCH = 4
SEQ = 2048
EMBED_DIM = 768
NUM_CORES = 2
NUM_SUBCORES = 16
NUM_WORKERS = NUM_CORES * NUM_SUBCORES
S_PER_W = SEQ // NUM_WORKERS
ROWS = 32
UNITS_PER_B = S_PER_W // ROWS
UNITS = BATCH * UNITS_PER_B
LANES = 16
VECS_PER_ROW = EMBED_DIM // LANES
NBUF = 3


def _embed_kernel(x_hbm, tok_hbm, pos_hbm, out_hbm,
                  idx_v, pos_v, tok0, tok1, tok2,
                  psem, g0, g1, g2, s0sem, s1sem, s2sem):
    wid = lax.axis_index("s") * NUM_CORES + lax.axis_index("c")
    s0 = wid * S_PER_W

    tok_bufs = (tok0, tok1, tok2)
    gsems = (g0, g1, g2)
    ssems = (s0sem, s1sem, s2sem)

    pos_cp = pltpu.async_copy(pos_hbm.at[pl.ds(s0, S_PER_W)], pos_v, psem)

    def gather(u):
        b, half = divmod(u, UNITS_PER_B)
        return pltpu.async_copy(
            tok_hbm.at[idx_v.at[b, pl.ds(half * ROWS, ROWS)]],
            tok_bufs[u % NBUF], gsems[u % NBUF])

    pltpu.sync_copy(x_hbm.at[0, pl.ds(s0, S_PER_W)], idx_v.at[0])
    gathers = [None] * UNITS
    stores = [None] * UNITS
    gathers[0] = gather(0)
    for b in range(1, BATCH):
        pltpu.sync_copy(x_hbm.at[b, pl.ds(s0, S_PER_W)], idx_v.at[b])
    pos_cp.wait()

    for u in range(UNITS):
        if u + 1 < UNITS:
            if u >= 2:
                stores[u - 2].wait()
            gathers[u + 1] = gather(u + 1)
        gathers[u].wait()

        buf = tok_bufs[u % NBUF]
        b, half = divmod(u, UNITS_PER_B)
        off = half * ROWS

        @plsc.parallel_loop(0, ROWS, 1, unroll=4)
        def add_row(r):
            for j in range(VECS_PER_ROW):
                sl = pl.ds(j * LANES, LANES)
                plsc.addupdate(buf.at[r, sl], pos_v[off + r, sl])

        stores[u] = pltpu.async_copy(
            buf, out_hbm.at[b, pl.ds(s0 + off, ROWS)], ssems[u % NBUF])

    for u in range(UNITS - 3, UNITS):
        stores[u].wait()


@jax.jit
def _embed(x, tok_table, pos_table):
    mesh = plsc.VectorSubcoreMesh(core_axis_name="c", subcore_axis_name="s")
    kfn = functools.partial(
        pl.kernel,
        mesh=mesh,
        out_type=jax.ShapeDtypeStruct((BATCH, SEQ, EMBED_DIM), jnp.float32),
        scratch_types=[
            pltpu.VMEM((BATCH, S_PER_W), jnp.int32),
            pltpu.VMEM((S_PER_W, EMBED_DIM), jnp.float32),
            pltpu.VMEM((ROWS, EMBED_DIM), jnp.float32),
            pltpu.VMEM((ROWS, EMBED_DIM), jnp.float32),
            pltpu.VMEM((ROWS, EMBED_DIM), jnp.float32),
            pltpu.SemaphoreType.DMA,
            pltpu.SemaphoreType.DMA,
            pltpu.SemaphoreType.DMA,
            pltpu.SemaphoreType.DMA,
            pltpu.SemaphoreType.DMA,
            pltpu.SemaphoreType.DMA,
            pltpu.SemaphoreType.DMA,
        ],
    )(_embed_kernel)
    return kfn(x, tok_table, pos_table)


def kernel(x, tok_table, pos_table):
    return _embed(x, tok_table, pos_table)


# speedup vs baseline: 1.0341x; 1.0341x over previous
import functools

import jax
import jax.numpy as jnp
from jax import lax
from jax.experimental import pallas as pl
from jax.experimental.pallas import tpu as pltpu
from jax.experimental.pallas import tpu_sc as plsc

BATCH = 4
SEQ = 2048
EMBED_DIM = 768
NUM_CORES = 2
NUM_SUBCORES = 16
NUM_WORKERS = NUM_CORES * NUM_SUBCORES
S_PER_W = SEQ // NUM_WORKERS
ROWS = 32
UNITS_PER_B = S_PER_W // ROWS
UNITS = BATCH * UNITS_PER_B
LANES = 16
VECS_PER_ROW = EMBED_DIM // LANES
NBUF = 3


def _embed_kernel(x_hbm, tok_hbm, pos_hbm, out_hbm,
                  idx_v, pos_v, tok0, tok1, tok2,
                  psem, g0, g1, g2, s0sem, s1sem, s2sem):
    wid = lax.axis_index("s") * NUM_CORES + lax.axis_index("c")
    s0 = wid * S_PER_W

    tok_bufs = (tok0, tok1, tok2)
    gsems = (g0, g1, g2)
    ssems = (s0sem, s1sem, s2sem)

    pos_cp = pltpu.async_copy(pos_hbm.at[pl.ds(s0, S_PER_W)], pos_v, psem)

    def gather(u):
        b, half = divmod(u, UNITS_PER_B)
        return pltpu.async_copy(
            tok_hbm.at[idx_v.at[b, pl.ds(half * ROWS, ROWS)]],
            tok_bufs[u % NBUF], gsems[u % NBUF])

    pltpu.sync_copy(x_hbm.at[0, pl.ds(s0, S_PER_W)], idx_v.at[0])
    gathers = [None] * UNITS
    stores = [None] * UNITS
    gathers[0] = gather(0)
    for b in range(1, BATCH):
        pltpu.sync_copy(x_hbm.at[b, pl.ds(s0, S_PER_W)], idx_v.at[b])
    gathers[1] = gather(1)
    pos_cp.wait()

    for u in range(UNITS):
        if u + 2 < UNITS:
            if u >= 1:
                stores[u - 1].wait()
            gathers[u + 2] = gather(u + 2)
        gathers[u].wait()

        buf = tok_bufs[u % NBUF]
        b, half = divmod(u, UNITS_PER_B)
        off = half * ROWS

        @plsc.parallel_loop(0, ROWS, 1, unroll=2)
        def add_row(r):
            for j in range(VECS_PER_ROW):
                sl = pl.ds(j * LANES, LANES)
                plsc.addupdate(buf.at[r, sl], pos_v[off + r, sl])

        stores[u] = pltpu.async_copy(
            buf, out_hbm.at[b, pl.ds(s0 + off, ROWS)], ssems[u % NBUF])

    for u in range(UNITS - 3, UNITS):
        stores[u].wait()


@jax.jit
def _embed(x, tok_table, pos_table):
    mesh = plsc.VectorSubcoreMesh(core_axis_name="c", subcore_axis_name="s")
    kfn = functools.partial(
        pl.kernel,
        mesh=mesh,
        out_type=jax.ShapeDtypeStruct((BATCH, SEQ, EMBED_DIM), jnp.float32),
        scratch_types=[
            pltpu.VMEM((BATCH, S_PER_W), jnp.int32),
            pltpu.VMEM((S_PER_W, EMBED_DIM), jnp.float32),
            pltpu.VMEM((ROWS, EMBED_DIM), jnp.float32),
            pltpu.VMEM((ROWS, EMBED_DIM), jnp.float32),
            pltpu.VMEM((ROWS, EMBED_DIM), jnp.float32),
            pltpu.SemaphoreType.DMA,
            pltpu.SemaphoreType.DMA,
            pltpu.SemaphoreType.DMA,
            pltpu.SemaphoreType.DMA,
            pltpu.SemaphoreType.DMA,
            pltpu.SemaphoreType.DMA,
            pltpu.SemaphoreType.DMA,
        ],
    )(_embed_kernel)
    return kfn(x, tok_table, pos_table)


def kernel(x, tok_table, pos_table):
    return _embed(x, tok_table, pos_table)


# speedup vs baseline: 1.0653x; 1.0301x over previous
import functools

import jax
import jax.numpy as jnp
from jax import lax
from jax.experimental import pallas as pl
from jax.experimental.pallas import tpu as pltpu
from jax.experimental.pallas import tpu_sc as plsc

BATCH = 4
SEQ = 2048
EMBED_DIM = 768
NUM_CORES = 2
NUM_SUBCORES = 16
NUM_WORKERS = NUM_CORES * NUM_SUBCORES
ROWS = 32
NSLICE = 2
SLICE_STRIDE = NUM_WORKERS * ROWS
UNITS = NSLICE * BATCH
LANES = 16
VECS_PER_ROW = EMBED_DIM // LANES
NBUF = 4
AHEAD = 2


def _embed_kernel(x_hbm, tok_hbm, pos_hbm, out_hbm, idx_v, pos_v, *rest):
    tok_bufs = rest[:NBUF]
    psem = rest[NBUF]
    gsems = rest[NBUF + 1:2 * NBUF + 1]
    ssems = rest[2 * NBUF + 1:]

    wid = lax.axis_index("s") * NUM_CORES + lax.axis_index("c")

    def sbase(u):
        return wid * ROWS + (u // BATCH) * SLICE_STRIDE

    pos_cp = pltpu.async_copy(pos_hbm.at[pl.ds(wid * ROWS, ROWS)], pos_v, psem)

    def copy_idx(u):
        pltpu.sync_copy(x_hbm.at[u % BATCH, pl.ds(sbase(u), ROWS)],
                        idx_v.at[u])

    def gather(u):
        return pltpu.async_copy(
            tok_hbm.at[idx_v.at[u]], tok_bufs[u % NBUF], gsems[u % NBUF])

    gathers = [None] * UNITS
    stores = [None] * UNITS
    copy_idx(0)
    gathers[0] = gather(0)
    copy_idx(1)
    gathers[1] = gather(1)
    for u in range(2, UNITS):
        copy_idx(u)
    pos_cp.wait()

    for u in range(UNITS):
        if u + AHEAD < UNITS:
            prev = u + AHEAD - NBUF
            if prev >= 0:
                stores[prev].wait()
            gathers[u + AHEAD] = gather(u + AHEAD)
        gathers[u].wait()
        if u == BATCH:
            pos_cp.wait()

        buf = tok_bufs[u % NBUF]

        @plsc.parallel_loop(0, ROWS, 1, unroll=2)
        def add_row(r):
            for j in range(VECS_PER_ROW):
                sl = pl.ds(j * LANES, LANES)
                plsc.addupdate(buf.at[r, sl], pos_v[r, sl])

        if u == BATCH - 1:
            pos_cp = pltpu.async_copy(
                pos_hbm.at[pl.ds(wid * ROWS + SLICE_STRIDE, ROWS)],
                pos_v, psem)

        stores[u] = pltpu.async_copy(
            buf, out_hbm.at[u % BATCH, pl.ds(sbase(u), ROWS)],
            ssems[u % NBUF])

    for u in range(max(0, UNITS - NBUF), UNITS):
        stores[u].wait()


@jax.jit
def _embed(x, tok_table, pos_table):
    mesh = plsc.VectorSubcoreMesh(core_axis_name="c", subcore_axis_name="s")
    kfn = functools.partial(
        pl.kernel,
        mesh=mesh,
        out_type=jax.ShapeDtypeStruct((BATCH, SEQ, EMBED_DIM), jnp.float32),
        scratch_types=(
            [pltpu.VMEM((UNITS, ROWS), jnp.int32),
             pltpu.VMEM((ROWS, EMBED_DIM), jnp.float32)]
            + [pltpu.VMEM((ROWS, EMBED_DIM), jnp.float32)] * NBUF
            + [pltpu.SemaphoreType.DMA] * (2 * NBUF + 1)
        ),
    )(_embed_kernel)
    return kfn(x, tok_table, pos_table)


def kernel(x, tok_table, pos_table):
    return _embed(x, tok_table, pos_table)


# speedup vs baseline: 1.0760x; 1.0101x over previous
import functools

import jax
import jax.numpy as jnp
from jax import lax
from jax.experimental import pallas as pl
from jax.experimental.pallas import tpu as pltpu
from jax.experimental.pallas import tpu_sc as plsc

BATCH = 4
SEQ = 2048
EMBED_DIM = 768
NUM_CORES = 2
NUM_SUBCORES = 16
NUM_WORKERS = NUM_CORES * NUM_SUBCORES
S_PER_W = SEQ // NUM_WORKERS
ROWS = 32
UNITS_PER_B = S_PER_W // ROWS
UNITS = BATCH * UNITS_PER_B
LANES = 16
VECS_PER_ROW = EMBED_DIM // LANES
NBUF = 3
AHEAD = 1


def _embed_kernel(x_hbm, tok_hbm, pos_hbm, out_hbm,
                  idx_v, pos_v, tok0, tok1, tok2,
                  psem, g0, g1, g2, s0sem, s1sem, s2sem):
    wid = lax.axis_index("s") * NUM_CORES + lax.axis_index("c")
    s0 = wid * S_PER_W

    tok_bufs = (tok0, tok1, tok2)
    gsems = (g0, g1, g2)
    ssems = (s0sem, s1sem, s2sem)

    pos_cp = pltpu.async_copy(pos_hbm.at[pl.ds(s0, S_PER_W)], pos_v, psem)

    def gather(u):
        b, half = divmod(u, UNITS_PER_B)
        return pltpu.async_copy(
            tok_hbm.at[idx_v.at[b, pl.ds(half * ROWS, ROWS)]],
            tok_bufs[u % NBUF], gsems[u % NBUF])

    pltpu.sync_copy(x_hbm.at[0, pl.ds(s0, S_PER_W)], idx_v.at[0])
    gathers = [None] * UNITS
    stores = [None] * UNITS
    gathers[0] = gather(0)
    for b in range(1, BATCH):
        pltpu.sync_copy(x_hbm.at[b, pl.ds(s0, S_PER_W)], idx_v.at[b])
    pos_cp.wait()

    for u in range(UNITS):
        if u + AHEAD < UNITS:
            prev = u + AHEAD - NBUF
            if prev >= 0:
                stores[prev].wait()
            gathers[u + AHEAD] = gather(u + AHEAD)
        gathers[u].wait()

        buf = tok_bufs[u % NBUF]
        b, half = divmod(u, UNITS_PER_B)
        off = half * ROWS

        @plsc.parallel_loop(0, ROWS, 1, unroll=2)
        def add_row(r):
            for j in range(VECS_PER_ROW):
                sl = pl.ds(j * LANES, LANES)
                plsc.addupdate(buf.at[r, sl], pos_v[off + r, sl])

        stores[u] = pltpu.async_copy(
            buf, out_hbm.at[b, pl.ds(s0 + off, ROWS)], ssems[u % NBUF])

    for u in range(max(0, UNITS - NBUF), UNITS):
        stores[u].wait()


@jax.jit
def _embed(x, tok_table, pos_table):
    mesh = plsc.VectorSubcoreMesh(core_axis_name="c", subcore_axis_name="s")
    kfn = functools.partial(
        pl.kernel,
        mesh=mesh,
        out_type=jax.ShapeDtypeStruct((BATCH, SEQ, EMBED_DIM), jnp.float32),
        scratch_types=[
            pltpu.VMEM((BATCH, S_PER_W), jnp.int32),
            pltpu.VMEM((S_PER_W, EMBED_DIM), jnp.float32),
            pltpu.VMEM((ROWS, EMBED_DIM), jnp.float32),
            pltpu.VMEM((ROWS, EMBED_DIM), jnp.float32),
            pltpu.VMEM((ROWS, EMBED_DIM), jnp.float32),
            pltpu.SemaphoreType.DMA,
            pltpu.SemaphoreType.DMA,
            pltpu.SemaphoreType.DMA,
            pltpu.SemaphoreType.DMA,
            pltpu.SemaphoreType.DMA,
            pltpu.SemaphoreType.DMA,
            pltpu.SemaphoreType.DMA,
        ],
    )(_embed_kernel)
    return kfn(x, tok_table, pos_table)


def kernel(x, tok_table, pos_table):
    return _embed(x, tok_table, pos_table)
